# baseline (speedup 1.0000x reference)
import functools

import jax
import jax.numpy as jnp
from jax import lax
from jax.experimental import pallas as pl
from jax.experimental.pallas import tpu as pltpu
from jax.experimental.pallas import tpu_sc as plsc

N = 10000
E = 160000
E_LG = 320000
D = 128

NC = 2
NS = 16
L = 16


_SEG = 2048
_SH = 18
_SLOT = 256


def _seg_params(B, V, G):
    n_slots = -(-V // _SLOT)
    NW = NC * NS
    max_spw = -(-n_slots // NW)
    CAP = -(-(B * max_spw // NW * 3 + max_spw * G) // 128) * 128 + 128
    assert B % _SEG == 0 and B < (1 << _SH)
    assert (_SLOT + 8) < (1 << (31 - _SH))
    return n_slots, max_spw, CAP


def _sc_seg_scan(idx, V, G):
    B = idx.shape[0]
    n_slots, max_spw, CAP = _seg_params(B, V, G)
    NSEG = B // _SEG
    SBLK = _SEG // 16
    OSLOTS = -(-(16 * (max_spw + 1)) // 128) * 128
    mesh = plsc.VectorSubcoreMesh(core_axis_name="c", subcore_axis_name="s",
                                  num_cores=NC, num_subcores=NS)

    @functools.partial(
        pl.kernel,
        out_type=[
            jax.ShapeDtypeStruct((NC * NS * CAP,), jnp.int32),
            jax.ShapeDtypeStruct((NC * NS * OSLOTS,), jnp.int32),
        ],
        mesh=mesh,
        compiler_params=pltpu.CompilerParams(needs_layout_passes=False),
        scratch_types=[
            pltpu.VMEM((_SEG,), jnp.int32),
            pltpu.VMEM((CAP,), jnp.int32),
            pltpu.VMEM((OSLOTS,), jnp.int32),
        ],
    )
    def scan_kernel(idx_hbm, sel_hbm, offs_hbm, seg_v, buf_v, offs_v):
        c = lax.axis_index("c")
        s = lax.axis_index("s")
        wid = c * NS + s
        iota = lax.iota(jnp.int32, L)
        off = 0
        for q in range(max_spw):
            slot = q * (NC * NS) + wid
            offs_v[pl.ds(q * 16, L)] = jnp.broadcast_to(off, (L,))
            base = slot * _SLOT

            def scan_seg(sg, off_s):
                pltpu.sync_copy(idx_hbm.at[pl.ds(sg * _SEG, _SEG)], seg_v)

                def scan_blk(b, o):
                    iv = seg_v[pl.ds(b * 16, 16)]
                    m = (iv >= base) & (iv < base + _SLOT)
                    mi = m.astype(jnp.int32)
                    pos = o + plsc.cumsum(mi) - mi
                    pack = ((iv - base) << _SH) | (sg * _SEG + b * 16 + iota)
                    plsc.store_scatter(buf_v, [pos], pack, mask=m)
                    return o + jnp.sum(mi)

                return lax.fori_loop(0, SBLK, scan_blk, off_s, unroll=False)

            valid = (slot < n_slots).astype(jnp.int32)
            n_sel = lax.fori_loop(0, NSEG * valid, scan_seg, off,
                                  unroll=False)
            nb_end = off + (n_sel - off + G - 1) // G * G
            for t in range(4):
                pos = n_sel + t * 16 + iota
                m = pos < nb_end
                plsc.store_scatter(buf_v, [pos],
                                   ((_SLOT + (iota & 7)) << _SH) | 0, mask=m)
            off = nb_end
        offs_v[pl.ds(max_spw * 16, L)] = jnp.broadcast_to(off, (L,))

        def flush(t, _):
            pltpu.sync_copy(buf_v.at[pl.ds(t * 128, 128)],
                            sel_hbm.at[pl.ds(wid * CAP + t * 128, 128)])
            return 0

        lax.fori_loop(0, (off + 127) // 128, flush, 0, unroll=False)
        pltpu.sync_copy(offs_v, offs_hbm.at[pl.ds(wid * OSLOTS, OSLOTS)])

    return scan_kernel(idx)


def _sc_seg_accum(vals, sel, offs, idx, V, G):
    W = vals.shape[1]
    B = idx.shape[0]
    n_slots, max_spw, CAP = _seg_params(B, V, G)
    Vpad = n_slots * _SLOT
    OSLOTS = -(-(16 * (max_spw + 1)) // 128) * 128
    mesh = plsc.VectorSubcoreMesh(core_axis_name="c", subcore_axis_name="s",
                                  num_cores=NC, num_subcores=NS)

    @functools.partial(
        pl.kernel,
        out_type=jax.ShapeDtypeStruct((Vpad, W), jnp.float32),
        mesh=mesh,
        compiler_params=pltpu.CompilerParams(needs_layout_passes=False),
        scratch_types=[
            pltpu.VMEM((OSLOTS,), jnp.int32),
            pltpu.VMEM((CAP,), jnp.int32),
            pltpu.VMEM((G,), jnp.int32),
            pltpu.VMEM((G + 16,), jnp.int32),
            pltpu.VMEM((G, W), jnp.float32),
            pltpu.VMEM((_SLOT + 16, W), jnp.float32),
            pltpu.SemaphoreType.DMA,
        ],
    )
    def accum_kernel(vals_hbm, sel_hbm, offs_hbm, out_hbm, offs_v, self_v,
                     eid_st, loc_st, rows_v, acc_v, sem):
        c = lax.axis_index("c")
        s = lax.axis_index("s")
        wid = c * NS + s
        iota = lax.iota(jnp.int32, L)
        pltpu.sync_copy(offs_hbm.at[pl.ds(wid * OSLOTS, OSLOTS)], offs_v)
        pltpu.sync_copy(sel_hbm.at[pl.ds(wid * CAP, CAP)], self_v)
        zeros = jnp.zeros((L,), jnp.float32)

        for q in range(max_spw):
            slot = q * (NC * NS) + wid
            base = slot * _SLOT

            @pl.when(slot < n_slots)
            def _slot_body():
                def zrow(r, t):
                    for ch in range(W // L):
                        acc_v[r, pl.ds(ch * L, L)] = zeros
                    return t

                lax.fori_loop(0, _SLOT + 16, zrow, 0, unroll=False)
                start = pl.multiple_of(jnp.max(offs_v[pl.ds(q * 16, L)]), G)
                end = jnp.max(offs_v[pl.ds((q + 1) * 16, L)])

                def batch(bi, _):
                    for ch in range(G // L):
                        v = self_v[pl.ds(start + bi * G + ch * L, L)]
                        eid_st[pl.ds(ch * L, L)] = v & ((1 << _SH) - 1)
                        loc_st[pl.ds(ch * L, L)] = v >> _SH
                    pltpu.async_copy(vals_hbm.at[eid_st], rows_v, sem).wait()

                    def addrow(r, t):
                        loc = loc_st[pl.ds(r, L)][0]
                        for ch in range(W // L):
                            plsc.addupdate(acc_v.at[loc, pl.ds(ch * L, L)],
                                           rows_v[r, pl.ds(ch * L, L)])
                        return t

                    lax.fori_loop(0, G, addrow, 0, unroll=False)
                    return 0

                lax.fori_loop(0, (end - start) // G, batch, 0, unroll=False)
                pltpu.sync_copy(acc_v.at[pl.ds(0, _SLOT)],
                                out_hbm.at[pl.ds(base, _SLOT)])

    return accum_kernel(vals, sel, offs)


def _sc_segment_sum(vals, idx, V, G=64):
    B = vals.shape[0]
    B2 = -(-B // _SEG) * _SEG
    if B2 != B:
        sentinel = -(-V // _SLOT) * _SLOT + 8
        idx = jnp.concatenate(
            [idx, jnp.full((B2 - B,), sentinel, jnp.int32)])
    sel, offs = _sc_seg_scan(idx, V, G)
    return _sc_seg_accum(vals, sel, offs, idx, V, G)


def _mm_tables_body(x_ref, w_ref, b_ref, a_ref, bu_ref, c_ref):
    t = jnp.dot(x_ref[...], w_ref[...], preferred_element_type=jnp.float32)
    t = t + b_ref[...]
    a_ref[...] = t[:, :D]
    bu_ref[...] = t[:, D:3 * D]
    c_ref[...] = t[:, 3 * D:]


def _mm_tables(x, Wcat, bcat, blk):
    V = x.shape[0]
    assert V % blk == 0
    return pl.pallas_call(
        _mm_tables_body,
        grid=(V // blk,),
        in_specs=[
            pl.BlockSpec((blk, D), lambda r: (r, 0)),
            pl.BlockSpec((D, 4 * D), lambda r: (0, 0)),
            pl.BlockSpec((1, 4 * D), lambda r: (0, 0)),
        ],
        out_specs=[
            pl.BlockSpec((blk, D), lambda r: (r, 0)),
            pl.BlockSpec((blk, 2 * D), lambda r: (r, 0)),
            pl.BlockSpec((blk, D), lambda r: (r, 0)),
        ],
        out_shape=[
            jax.ShapeDtypeStruct((V, D), jnp.float32),
            jax.ShapeDtypeStruct((V, 2 * D), jnp.float32),
            jax.ShapeDtypeStruct((V, D), jnp.float32),
        ],
    )(x, Wcat, bcat.reshape(1, 4 * D))


def _mm_bias_body(x_ref, w_ref, b_ref, o_ref):
    o_ref[...] = jnp.dot(x_ref[...], w_ref[...],
                         preferred_element_type=jnp.float32) + b_ref[...]


def _mm_bias(x, W, b, blk):
    V = x.shape[0]
    assert V % blk == 0
    return pl.pallas_call(
        _mm_bias_body,
        grid=(V // blk,),
        in_specs=[
            pl.BlockSpec((blk, D), lambda r: (r, 0)),
            pl.BlockSpec((D, D), lambda r: (0, 0)),
            pl.BlockSpec((1, D), lambda r: (0, 0)),
        ],
        out_specs=pl.BlockSpec((blk, D), lambda r: (r, 0)),
        out_shape=jax.ShapeDtypeStruct((V, D), jnp.float32),
    )(x, W, b.reshape(1, D))


def _col_stats_body(t_ref, o_ref):
    blk = t_ref[...]
    s = jnp.sum(blk, axis=0, keepdims=True)
    q = jnp.sum(blk * blk, axis=0, keepdims=True)
    upd = jnp.concatenate([s, q, jnp.zeros((6, D), jnp.float32)], axis=0)

    @pl.when(pl.program_id(0) == 0)
    def _():
        o_ref[...] = jnp.zeros_like(o_ref)

    o_ref[...] += upd


def _col_stats(t, blk):
    V = t.shape[0]
    assert V % blk == 0
    return pl.pallas_call(
        _col_stats_body,
        grid=(V // blk,),
        in_specs=[pl.BlockSpec((blk, D), lambda r: (r, 0))],
        out_specs=pl.BlockSpec((8, D), lambda r: (0, 0)),
        out_shape=jax.ShapeDtypeStruct((8, D), jnp.float32),
    )(t)


def _bn_apply_body(count, base_ref, t_ref, st_ref, g_ref, b_ref, o_ref):
    s = st_ref[0, :]
    q = st_ref[1, :]
    mean = s / count
    var = q / count - mean * mean
    rstd = jax.lax.rsqrt(var + 1e-5)
    h = (t_ref[...] - mean) * (rstd * g_ref[...]) + b_ref[...]
    o_ref[...] = base_ref[...] + h / (1.0 + jnp.exp(-h))


def _bn_apply_residual(base, t, stats, gamma, beta, blk):
    V = t.shape[0]
    assert V % blk == 0
    t2 = t.reshape(V, D) if t.ndim == 2 else t
    return pl.pallas_call(
        functools.partial(_bn_apply_body, float(V)),
        grid=(V // blk,),
        in_specs=[
            pl.BlockSpec((blk, D), lambda r: (r, 0)),
            pl.BlockSpec((blk, D), lambda r: (r, 0)),
            pl.BlockSpec((8, D), lambda r: (0, 0)),
            pl.BlockSpec((1, D), lambda r: (0, 0)),
            pl.BlockSpec((1, D), lambda r: (0, 0)),
        ],
        out_specs=pl.BlockSpec((blk, D), lambda r: (r, 0)),
        out_shape=jax.ShapeDtypeStruct((V, D), jnp.float32),
    )(base, t2, stats, gamma.reshape(1, D), beta.reshape(1, D))


def _egc(edge_index, x, edge_attr, W, b, bn_g, bn_b, num_nodes,
         node_blk, edge_blk, use_sc_seg):
    i = edge_index[0]
    j = edge_index[1]
    Wcat = jnp.concatenate([W[0], W[1], W[4], W[3]], axis=1)
    bcat = jnp.concatenate([b[0], b[1], b[4], b[3]], axis=0)
    A, BU, C = _mm_tables(x, Wcat, bcat, node_blk)
    EY = _mm_bias(edge_attr, W[2], b[2], edge_blk)

    buj = jnp.take(BU, j, axis=0)
    em = jnp.take(A, i, axis=0) + buj[:, :D] + EY
    se = jax.nn.sigmoid(em)
    nm = se * buj[:, D:]
    if use_sc_seg:
        snm = jnp.concatenate([se, nm], axis=1)
        ssn = _sc_segment_sum(snm, i, num_nodes)
        SS = ssn[:num_nodes, :D]
        SN = ssn[:num_nodes, D:]
    else:
        SS = jax.ops.segment_sum(se, i, num_segments=num_nodes)
        SN = jax.ops.segment_sum(nm, i, num_segments=num_nodes)
    v = C + SN / (SS + 1e-9)

    em_stats = _col_stats(em, edge_blk)
    v_stats = _col_stats(v, node_blk)
    x_new = _bn_apply_residual(x, v, v_stats, bn_g[1], bn_b[1], node_blk)
    e_new = _bn_apply_residual(edge_attr, em, em_stats, bn_g[0], bn_b[0],
                               edge_blk)
    return x_new, e_new


def kernel(g, lg, x, y, z, W1, b1, bn1_g, bn1_b, W2, b2, bn2_g, bn2_b):
    x_new, m = _egc(g, x, y, W1, b1, bn1_g, bn1_b, N,
                    node_blk=400, edge_blk=1600, use_sc_seg=True)
    y_new, z_new = _egc(lg, m, z, W2, b2, bn2_g, bn2_b, E,
                        node_blk=1600, edge_blk=1600, use_sc_seg=False)
    return (x_new, y_new, z_new)
